# baseline (speedup 1.0000x reference)
import functools

import jax
import jax.numpy as jnp
from jax import lax
from jax.experimental import pallas as pl
from jax.experimental.pallas import tpu as pltpu
from jax.experimental.pallas import tpu_sc as plsc

N = 10000
E = 160000
D = 256
IN = 4
HALF = 128
NSUB = 16
K = 80
EPW = E // NSUB
RPS = 624


def _dotT(a, w):
    return lax.dot_general(a, w, (((1,), (1,)), ((), ())),
                           preferred_element_type=jnp.float32)


def _gelu(t):
    return 0.5 * t * (1.0 + lax.erf(t * (2.0 ** -0.5)))


def _sc_body(nbr_lo, nbr_hi, row_hbm, col_hbm, ev_hbm, out_lo, out_hi,
             row_all, col_va, ev_va, msg_a, col_vb, ev_vb, msg_b,
             acc, gsem_a, gsem_b, ssem_a, ssem_b, cisem_a, cisem_b, isem):
    c = lax.axis_index("c")
    s = lax.axis_index("s")

    d1 = pltpu.async_copy(row_hbm.at[s], row_all, isem)

    def zrow(i, carry):
        for j in range(HALF // 16):
            msg_a[i, pl.ds(j * 16, 16)] = jnp.zeros((16,), jnp.float32)
        return carry
    lax.fori_loop(0, K, zrow, 0)
    for t in range(RPS // K):
        pltpu.sync_copy(msg_a, acc.at[pl.ds(s * RPS + t * K, K)])
    rem = RPS - (RPS // K) * K
    if rem:
        pltpu.sync_copy(msg_a.at[pl.ds(0, rem)],
                        acc.at[pl.ds(s * RPS + (RPS // K) * K, rem)])

    @pl.when(s == NSUB - 1)
    def _():
        pltpu.sync_copy(msg_a.at[pl.ds(0, N - NSUB * RPS)],
                        acc.at[pl.ds(NSUB * RPS, N - NSUB * RPS)])
    d1.wait()
    plsc.subcore_barrier()

    set_a = (msg_a, col_va, ev_va, gsem_a, ssem_a, cisem_a)
    set_b = (msg_b, col_vb, ev_vb, gsem_b, ssem_b, cisem_b)
    CH = EPW // K

    def edge_sweep(nbr_ref):
        def start_prefetch(m, st):
            base = s * EPW + m * K
            pltpu.async_copy(col_hbm.at[pl.ds(base, K)], st[1], st[5])
            pltpu.async_copy(ev_hbm.at[pl.ds(base, K)], st[2], st[5])

        def wait_prefetch(m, st):
            base = s * EPW + m * K
            pltpu.make_async_copy(col_hbm.at[pl.ds(base, K)], st[1], st[5]).wait()
            pltpu.make_async_copy(ev_hbm.at[pl.ds(base, K)], st[2], st[5]).wait()

        def start_gather(m, st):
            pltpu.async_copy(nbr_ref.at[st[1]], st[0], st[3])

        def wait_gather(m, st):
            pltpu.make_async_copy(nbr_ref.at[st[1]], st[0], st[3]).wait()

        def start_scatter(m, st):
            pltpu.async_copy(st[0], acc.at[row_all.at[m]], st[4], add=True)

        def wait_scatter(m, st):
            pltpu.make_async_copy(st[0], acc.at[row_all.at[m]], st[4]).wait()

        def scale(m, st):
            msg_x, ev_x = st[0], st[2]

            @plsc.parallel_loop(0, K // 16)
            def _(g):
                ev16 = ev_x[pl.ds(g * 16, 16)]
                for l in range(16):
                    e = g * 16 + l
                    ev_s = ev16[l]
                    for j in range(HALF // 16):
                        msg_x[e, pl.ds(j * 16, 16)] = msg_x[e, pl.ds(j * 16, 16)] * ev_s

        def guarded(m, cond_limit, work):
            if isinstance(m, int):
                if m <= cond_limit:
                    work()
            else:
                pl.when(m <= cond_limit)(work)

        def phase(m, cur, nxt, first=False):
            wait_gather(m, cur)
            scale(m, cur)
            if not first:
                wait_scatter(m - 1, nxt)

            def nxt_work():
                wait_prefetch(m + 1, nxt)
                start_gather(m + 1, nxt)
            guarded(m + 1, CH - 1, nxt_work)
            start_scatter(m, cur)
            guarded(m + 2, CH - 1, lambda: start_prefetch(m + 2, cur))

        start_prefetch(0, set_a)
        start_prefetch(1, set_b)
        wait_prefetch(0, set_a)
        start_gather(0, set_a)
        phase(0, set_a, set_b, first=True)

        def body(i, carry):
            phase(2 * i + 1, set_b, set_a)
            phase(2 * i + 2, set_a, set_b)
            return carry
        lax.fori_loop(0, (CH - 1) // 2, body, 0)
        wait_scatter(CH - 1, set_a if (CH - 1) % 2 == 0 else set_b)

    @pl.when(c == 0)
    def _():
        edge_sweep(nbr_lo)

    @pl.when(c == 1)
    def _():
        edge_sweep(nbr_hi)

    plsc.subcore_barrier()

    def writeback(out_ref):
        pltpu.sync_copy(acc.at[pl.ds(s * RPS, RPS)], out_ref.at[pl.ds(s * RPS, RPS)])

        @pl.when(s == NSUB - 1)
        def _():
            pltpu.sync_copy(acc.at[pl.ds(NSUB * RPS, N - NSUB * RPS)],
                            out_ref.at[pl.ds(NSUB * RPS, N - NSUB * RPS)])

    @pl.when(c == 0)
    def _():
        writeback(out_lo)

    @pl.when(c == 1)
    def _():
        writeback(out_hi)


@functools.cache
def _get_sc_aggregate():
  return pl.kernel(
    _sc_body,
    out_type=(jax.ShapeDtypeStruct((N, HALF), jnp.float32),
              jax.ShapeDtypeStruct((N, HALF), jnp.float32)),
    mesh=plsc.VectorSubcoreMesh(core_axis_name="c", subcore_axis_name="s"),
    scratch_types=[
        pltpu.VMEM((EPW // K, K), jnp.int32),
        pltpu.VMEM((K,), jnp.int32),
        pltpu.VMEM((K,), jnp.float32),
        pltpu.VMEM((K, HALF), jnp.float32),
        pltpu.VMEM((K,), jnp.int32),
        pltpu.VMEM((K,), jnp.float32),
        pltpu.VMEM((K, HALF), jnp.float32),
        pltpu.VMEM_SHARED((N, HALF), jnp.float32),
        pltpu.SemaphoreType.DMA,
        pltpu.SemaphoreType.DMA,
        pltpu.SemaphoreType.DMA,
        pltpu.SemaphoreType.DMA,
        pltpu.SemaphoreType.DMA,
        pltpu.SemaphoreType.DMA,
        pltpu.SemaphoreType.DMA,
    ],
  )


R = 1000
GRID = N // R


def _rows(width):
    return pl.BlockSpec((R, width), lambda i: (i, 0))


def _full(shape):
    return pl.BlockSpec(shape, lambda i: (0,) * len(shape))


def _tc_a_body(x_ref, w1, b1, w2, b2, wn, bn, ws, bs,
               h_ref, nl_ref, nh_ref, sf_ref):
    t = _gelu(_dotT(x_ref[...], w1[...]) + b1[...])
    h = _dotT(t, w2[...]) + b2[...]
    h_ref[...] = h
    nbr = _dotT(h, wn[...]) + bn[...]
    nl_ref[...] = nbr[:, :HALF]
    nh_ref[...] = nbr[:, HALF:]
    sf_ref[...] = _dotT(h, ws[...]) + bs[...]


_tc_a = pl.pallas_call(
    _tc_a_body,
    grid=(GRID,),
    in_specs=[_rows(IN), _full((D, IN)), _full((1, D)), _full((D, D)),
              _full((1, D)), _full((D, D)), _full((1, D)), _full((D, D)),
              _full((1, D))],
    out_specs=[_rows(D), _rows(HALF), _rows(HALF), _rows(D)],
    out_shape=[jax.ShapeDtypeStruct((N, D), jnp.float32),
               jax.ShapeDtypeStruct((N, HALF), jnp.float32),
               jax.ShapeDtypeStruct((N, HALF), jnp.float32),
               jax.ShapeDtypeStruct((N, D), jnp.float32)],
)


def _tc_b_body(h_ref, sf_ref, al_ref, ah_ref, wg1s, wg1l, wg1h, bg1, wg2, bg2,
               wn, bn, ws, bs, h1_ref, nl_ref, nh_ref, s1_ref):
    t = (_dotT(sf_ref[...], wg1s[...]) + _dotT(al_ref[...], wg1l[...])
         + _dotT(ah_ref[...], wg1h[...]) + bg1[...])
    out = _dotT(_gelu(t), wg2[...]) + bg2[...]
    h1 = h_ref[...] + out
    h1_ref[...] = h1
    nbr = _dotT(h1, wn[...]) + bn[...]
    nl_ref[...] = nbr[:, :HALF]
    nh_ref[...] = nbr[:, HALF:]
    s1_ref[...] = _dotT(h1, ws[...]) + bs[...]


_tc_b = pl.pallas_call(
    _tc_b_body,
    grid=(GRID,),
    in_specs=[_rows(D), _rows(D), _rows(HALF), _rows(HALF),
              _full((D, D)), _full((D, HALF)), _full((D, HALF)), _full((1, D)),
              _full((D, D)), _full((1, D)),
              _full((D, D)), _full((1, D)), _full((D, D)), _full((1, D))],
    out_specs=[_rows(D), _rows(HALF), _rows(HALF), _rows(D)],
    out_shape=[jax.ShapeDtypeStruct((N, D), jnp.float32),
               jax.ShapeDtypeStruct((N, HALF), jnp.float32),
               jax.ShapeDtypeStruct((N, HALF), jnp.float32),
               jax.ShapeDtypeStruct((N, D), jnp.float32)],
)


def _tc_c_body(h_ref, sf_ref, al_ref, ah_ref, wg1s, wg1l, wg1h, bg1, wg2, bg2,
               g_ref, bnorm_ref, o_ref):
    t = (_dotT(sf_ref[...], wg1s[...]) + _dotT(al_ref[...], wg1l[...])
         + _dotT(ah_ref[...], wg1h[...]) + bg1[...])
    out = _dotT(_gelu(t), wg2[...]) + bg2[...]
    hf = h_ref[...] + out
    mu = jnp.mean(hf, axis=-1, keepdims=True)
    var = jnp.mean((hf - mu) ** 2, axis=-1, keepdims=True)
    o_ref[...] = (hf - mu) / jnp.sqrt(var + 1e-5) * g_ref[...] + bnorm_ref[...]


_tc_c = pl.pallas_call(
    _tc_c_body,
    grid=(GRID,),
    in_specs=[_rows(D), _rows(D), _rows(HALF), _rows(HALF),
              _full((D, D)), _full((D, HALF)), _full((D, HALF)), _full((1, D)),
              _full((D, D)), _full((1, D)),
              _full((1, D)), _full((1, D))],
    out_specs=[_rows(D)],
    out_shape=[jax.ShapeDtypeStruct((N, D), jnp.float32)],
)


def kernel(x, edge_index, edge_values,
           lift_W1, lift_b1, lift_W2, lift_b2,
           gcn0_Ws, gcn0_bs, gcn0_Wn, gcn0_bn, gcn0_Wg1, gcn0_bg1, gcn0_Wg2, gcn0_bg2,
           gcn1_Ws, gcn1_bs, gcn1_Wn, gcn1_bn, gcn1_Wg1, gcn1_bg1, gcn1_Wg2, gcn1_bg2,
           norm_g, norm_b):
    x2 = x.reshape(N, IN)
    row = edge_index[0].reshape(NSUB, EPW // K, K)
    col = edge_index[1]

    def b(v):
        return v.reshape(1, D)

    h, nl, nh, sf = _tc_a(x2, lift_W1, b(lift_b1), lift_W2, b(lift_b2),
                          gcn0_Wn, b(gcn0_bn), gcn0_Ws, b(gcn0_bs))
    sc_aggregate = _get_sc_aggregate()
    al0, ah0 = sc_aggregate(nl, nh, row, col, edge_values)
    h1, nl1, nh1, s1 = _tc_b(h, sf, al0, ah0,
                             gcn0_Wg1[:, :D], gcn0_Wg1[:, D:D + HALF],
                             gcn0_Wg1[:, D + HALF:], b(gcn0_bg1),
                             gcn0_Wg2, b(gcn0_bg2),
                             gcn1_Wn, b(gcn1_bn), gcn1_Ws, b(gcn1_bs))
    al1, ah1 = sc_aggregate(nl1, nh1, row, col, edge_values)
    (out,) = _tc_c(h1, s1, al1, ah1,
                   gcn1_Wg1[:, :D], gcn1_Wg1[:, D:D + HALF],
                   gcn1_Wg1[:, D + HALF:], b(gcn1_bg1),
                   gcn1_Wg2, b(gcn1_bg2),
                   norm_g.reshape(1, D), norm_b.reshape(1, D))
    return out.reshape(1, N, D)


# speedup vs baseline: 1.2655x; 1.2655x over previous
import functools

import jax
import jax.numpy as jnp
from jax import lax
from jax.experimental import pallas as pl
from jax.experimental.pallas import tpu as pltpu
from jax.experimental.pallas import tpu_sc as plsc

N = 10000
E = 160000
D = 256
IN = 4
HALF = 128
NSUB = 16
K = 80
EPW = E // NSUB
RPS = 624


def _dotT(a, w):
    return lax.dot_general(a, w, (((1,), (1,)), ((), ())),
                           preferred_element_type=jnp.float32)


def _gelu(t):
    return 0.5 * t * (1.0 + lax.erf(t * (2.0 ** -0.5)))


def _sc_body(nbr_lo, nbr_hi, row_hbm, col_hbm, ev_hbm, out_lo, out_hi,
             row_all, col_va, ev_va, msg_a, col_vb, ev_vb, msg_b,
             col_vc, ev_vc, msg_c,
             acc, gsem_a, gsem_b, gsem_c, ssem_a, ssem_b, ssem_c,
             cisem_a, cisem_b, cisem_c, isem):
    c = lax.axis_index("c")
    s = lax.axis_index("s")

    d1 = pltpu.async_copy(row_hbm.at[s], row_all, isem)

    def zrow(i, carry):
        for j in range(HALF // 16):
            msg_a[i, pl.ds(j * 16, 16)] = jnp.zeros((16,), jnp.float32)
        return carry
    lax.fori_loop(0, K, zrow, 0)
    for t in range(RPS // K):
        pltpu.sync_copy(msg_a, acc.at[pl.ds(s * RPS + t * K, K)])
    rem = RPS - (RPS // K) * K
    if rem:
        pltpu.sync_copy(msg_a.at[pl.ds(0, rem)],
                        acc.at[pl.ds(s * RPS + (RPS // K) * K, rem)])

    @pl.when(s == NSUB - 1)
    def _():
        pltpu.sync_copy(msg_a.at[pl.ds(0, N - NSUB * RPS)],
                        acc.at[pl.ds(NSUB * RPS, N - NSUB * RPS)])
    d1.wait()
    plsc.subcore_barrier()

    sets = ((msg_a, col_va, ev_va, gsem_a, ssem_a, cisem_a),
            (msg_b, col_vb, ev_vb, gsem_b, ssem_b, cisem_b),
            (msg_c, col_vc, ev_vc, gsem_c, ssem_c, cisem_c))
    CH = EPW // K

    def edge_sweep(nbr_ref):
        def start_prefetch(m, st):
            base = s * EPW + m * K
            pltpu.async_copy(col_hbm.at[pl.ds(base, K)], st[1], st[5])
            pltpu.async_copy(ev_hbm.at[pl.ds(base, K)], st[2], st[5])

        def wait_prefetch(m, st):
            base = s * EPW + m * K
            pltpu.make_async_copy(col_hbm.at[pl.ds(base, K)], st[1], st[5]).wait()
            pltpu.make_async_copy(ev_hbm.at[pl.ds(base, K)], st[2], st[5]).wait()

        def start_gather(m, st):
            pltpu.async_copy(nbr_ref.at[st[1]], st[0], st[3])

        def wait_gather(m, st):
            pltpu.make_async_copy(nbr_ref.at[st[1]], st[0], st[3]).wait()

        def start_scatter(m, st):
            pltpu.async_copy(st[0], acc.at[row_all.at[m]], st[4], add=True)

        def wait_scatter(m, st):
            pltpu.make_async_copy(st[0], acc.at[row_all.at[m]], st[4]).wait()

        def scale(m, st):
            msg_x, ev_x = st[0], st[2]

            @plsc.parallel_loop(0, K // 16)
            def _(g):
                ev16 = ev_x[pl.ds(g * 16, 16)]
                for l in range(16):
                    e = g * 16 + l
                    ev_s = ev16[l]
                    for j in range(HALF // 16):
                        msg_x[e, pl.ds(j * 16, 16)] = msg_x[e, pl.ds(j * 16, 16)] * ev_s

        def guarded(m, cond_limit, work):
            if isinstance(m, int):
                if m <= cond_limit:
                    work()
            else:
                pl.when(m <= cond_limit)(work)

        def phase(m, cur, nxt, first=False):
            if not first:
                wait_scatter(m - 2, nxt)

            def nxt_work():
                wait_prefetch(m + 1, nxt)
                start_gather(m + 1, nxt)
            guarded(m + 1, CH - 1, nxt_work)
            wait_gather(m, cur)
            scale(m, cur)
            start_scatter(m, cur)
            guarded(m + 3, CH - 1, lambda: start_prefetch(m + 3, cur))

        start_prefetch(0, sets[0])
        start_prefetch(1, sets[1])
        start_prefetch(2, sets[2])
        wait_prefetch(0, sets[0])
        start_gather(0, sets[0])
        phase(0, sets[0], sets[1], first=True)
        phase(1, sets[1], sets[2], first=True)

        def body(i, carry):
            m0 = 3 * i + 2
            phase(m0, sets[2], sets[0])
            phase(m0 + 1, sets[0], sets[1])
            phase(m0 + 2, sets[1], sets[2])
            return carry
        lax.fori_loop(0, (CH - 2) // 3, body, 0)
        wait_scatter(CH - 2, sets[(CH - 2) % 3])
        wait_scatter(CH - 1, sets[(CH - 1) % 3])

    @pl.when(c == 0)
    def _():
        edge_sweep(nbr_lo)

    @pl.when(c == 1)
    def _():
        edge_sweep(nbr_hi)

    plsc.subcore_barrier()

    def writeback(out_ref):
        pltpu.sync_copy(acc.at[pl.ds(s * RPS, RPS)], out_ref.at[pl.ds(s * RPS, RPS)])

        @pl.when(s == NSUB - 1)
        def _():
            pltpu.sync_copy(acc.at[pl.ds(NSUB * RPS, N - NSUB * RPS)],
                            out_ref.at[pl.ds(NSUB * RPS, N - NSUB * RPS)])

    @pl.when(c == 0)
    def _():
        writeback(out_lo)

    @pl.when(c == 1)
    def _():
        writeback(out_hi)


@functools.cache
def _get_sc_aggregate():
  return pl.kernel(
    _sc_body,
    out_type=(jax.ShapeDtypeStruct((N, HALF), jnp.float32),
              jax.ShapeDtypeStruct((N, HALF), jnp.float32)),
    mesh=plsc.VectorSubcoreMesh(core_axis_name="c", subcore_axis_name="s"),
    scratch_types=[
        pltpu.VMEM((EPW // K, K), jnp.int32),
        pltpu.VMEM((K,), jnp.int32),
        pltpu.VMEM((K,), jnp.float32),
        pltpu.VMEM((K, HALF), jnp.float32),
        pltpu.VMEM((K,), jnp.int32),
        pltpu.VMEM((K,), jnp.float32),
        pltpu.VMEM((K, HALF), jnp.float32),
        pltpu.VMEM((K,), jnp.int32),
        pltpu.VMEM((K,), jnp.float32),
        pltpu.VMEM((K, HALF), jnp.float32),
        pltpu.VMEM_SHARED((N, HALF), jnp.float32),
        pltpu.SemaphoreType.DMA,
        pltpu.SemaphoreType.DMA,
        pltpu.SemaphoreType.DMA,
        pltpu.SemaphoreType.DMA,
        pltpu.SemaphoreType.DMA,
        pltpu.SemaphoreType.DMA,
        pltpu.SemaphoreType.DMA,
        pltpu.SemaphoreType.DMA,
        pltpu.SemaphoreType.DMA,
        pltpu.SemaphoreType.DMA,
    ],
  )


R = 1000
GRID = N // R


def _rows(width):
    return pl.BlockSpec((R, width), lambda i: (i, 0))


def _full(shape):
    return pl.BlockSpec(shape, lambda i: (0,) * len(shape))


def _tc_a_body(x_ref, w1, b1, w2, b2, wn, bn, ws, bs,
               h_ref, nl_ref, nh_ref, sf_ref):
    t = _gelu(_dotT(x_ref[...], w1[...]) + b1[...])
    h = _dotT(t, w2[...]) + b2[...]
    h_ref[...] = h
    nbr = _dotT(h, wn[...]) + bn[...]
    nl_ref[...] = nbr[:, :HALF]
    nh_ref[...] = nbr[:, HALF:]
    sf_ref[...] = _dotT(h, ws[...]) + bs[...]


_tc_a = pl.pallas_call(
    _tc_a_body,
    grid=(GRID,),
    in_specs=[_rows(IN), _full((D, IN)), _full((1, D)), _full((D, D)),
              _full((1, D)), _full((D, D)), _full((1, D)), _full((D, D)),
              _full((1, D))],
    out_specs=[_rows(D), _rows(HALF), _rows(HALF), _rows(D)],
    out_shape=[jax.ShapeDtypeStruct((N, D), jnp.float32),
               jax.ShapeDtypeStruct((N, HALF), jnp.float32),
               jax.ShapeDtypeStruct((N, HALF), jnp.float32),
               jax.ShapeDtypeStruct((N, D), jnp.float32)],
)


def _tc_b_body(h_ref, sf_ref, al_ref, ah_ref, wg1s, wg1l, wg1h, bg1, wg2, bg2,
               wn, bn, ws, bs, h1_ref, nl_ref, nh_ref, s1_ref):
    t = (_dotT(sf_ref[...], wg1s[...]) + _dotT(al_ref[...], wg1l[...])
         + _dotT(ah_ref[...], wg1h[...]) + bg1[...])
    out = _dotT(_gelu(t), wg2[...]) + bg2[...]
    h1 = h_ref[...] + out
    h1_ref[...] = h1
    nbr = _dotT(h1, wn[...]) + bn[...]
    nl_ref[...] = nbr[:, :HALF]
    nh_ref[...] = nbr[:, HALF:]
    s1_ref[...] = _dotT(h1, ws[...]) + bs[...]


_tc_b = pl.pallas_call(
    _tc_b_body,
    grid=(GRID,),
    in_specs=[_rows(D), _rows(D), _rows(HALF), _rows(HALF),
              _full((D, D)), _full((D, HALF)), _full((D, HALF)), _full((1, D)),
              _full((D, D)), _full((1, D)),
              _full((D, D)), _full((1, D)), _full((D, D)), _full((1, D))],
    out_specs=[_rows(D), _rows(HALF), _rows(HALF), _rows(D)],
    out_shape=[jax.ShapeDtypeStruct((N, D), jnp.float32),
               jax.ShapeDtypeStruct((N, HALF), jnp.float32),
               jax.ShapeDtypeStruct((N, HALF), jnp.float32),
               jax.ShapeDtypeStruct((N, D), jnp.float32)],
)


def _tc_c_body(h_ref, sf_ref, al_ref, ah_ref, wg1s, wg1l, wg1h, bg1, wg2, bg2,
               g_ref, bnorm_ref, o_ref):
    t = (_dotT(sf_ref[...], wg1s[...]) + _dotT(al_ref[...], wg1l[...])
         + _dotT(ah_ref[...], wg1h[...]) + bg1[...])
    out = _dotT(_gelu(t), wg2[...]) + bg2[...]
    hf = h_ref[...] + out
    mu = jnp.mean(hf, axis=-1, keepdims=True)
    var = jnp.mean((hf - mu) ** 2, axis=-1, keepdims=True)
    o_ref[...] = (hf - mu) / jnp.sqrt(var + 1e-5) * g_ref[...] + bnorm_ref[...]


_tc_c = pl.pallas_call(
    _tc_c_body,
    grid=(GRID,),
    in_specs=[_rows(D), _rows(D), _rows(HALF), _rows(HALF),
              _full((D, D)), _full((D, HALF)), _full((D, HALF)), _full((1, D)),
              _full((D, D)), _full((1, D)),
              _full((1, D)), _full((1, D))],
    out_specs=[_rows(D)],
    out_shape=[jax.ShapeDtypeStruct((N, D), jnp.float32)],
)


def kernel(x, edge_index, edge_values,
           lift_W1, lift_b1, lift_W2, lift_b2,
           gcn0_Ws, gcn0_bs, gcn0_Wn, gcn0_bn, gcn0_Wg1, gcn0_bg1, gcn0_Wg2, gcn0_bg2,
           gcn1_Ws, gcn1_bs, gcn1_Wn, gcn1_bn, gcn1_Wg1, gcn1_bg1, gcn1_Wg2, gcn1_bg2,
           norm_g, norm_b):
    x2 = x.reshape(N, IN)
    row = edge_index[0].reshape(NSUB, EPW // K, K)
    col = edge_index[1]

    def b(v):
        return v.reshape(1, D)

    h, nl, nh, sf = _tc_a(x2, lift_W1, b(lift_b1), lift_W2, b(lift_b2),
                          gcn0_Wn, b(gcn0_bn), gcn0_Ws, b(gcn0_bs))
    sc_aggregate = _get_sc_aggregate()
    al0, ah0 = sc_aggregate(nl, nh, row, col, edge_values)
    h1, nl1, nh1, s1 = _tc_b(h, sf, al0, ah0,
                             gcn0_Wg1[:, :D], gcn0_Wg1[:, D:D + HALF],
                             gcn0_Wg1[:, D + HALF:], b(gcn0_bg1),
                             gcn0_Wg2, b(gcn0_bg2),
                             gcn1_Wn, b(gcn1_bn), gcn1_Ws, b(gcn1_bs))
    al1, ah1 = sc_aggregate(nl1, nh1, row, col, edge_values)
    (out,) = _tc_c(h1, s1, al1, ah1,
                   gcn1_Wg1[:, :D], gcn1_Wg1[:, D:D + HALF],
                   gcn1_Wg1[:, D + HALF:], b(gcn1_bg1),
                   gcn1_Wg2, b(gcn1_bg2),
                   norm_g.reshape(1, D), norm_b.reshape(1, D))
    return out.reshape(1, N, D)


# speedup vs baseline: 1.3563x; 1.0718x over previous
import functools

import jax
import jax.numpy as jnp
from jax import lax
from jax.experimental import pallas as pl
from jax.experimental.pallas import tpu as pltpu
from jax.experimental.pallas import tpu_sc as plsc

N = 10000
E = 160000
D = 256
IN = 4
HALF = 128
NSUB = 16
K = 80
EPW = E // NSUB
CH = EPW // K
NBUF = 4
RPS = 624


def _dotT(a, w):
    return lax.dot_general(a, w, (((1,), (1,)), ((), ())),
                           preferred_element_type=jnp.float32)


def _gelu(t):
    return 0.5 * t * (1.0 + lax.erf(t * (2.0 ** -0.5)))


def _sc_body(nbr2, row_hbm, col_hbm, ev_hbm, out2, *rest):
    bufs, (acc,), sems = rest[:4 * NBUF], rest[4 * NBUF:4 * NBUF + 1], rest[4 * NBUF + 1:]
    sets = tuple(bufs[4 * i:4 * i + 4] + sems[4 * i:4 * i + 4]
                 for i in range(NBUF))
    c = lax.axis_index("c")
    s = lax.axis_index("s")
    nbr_ref = nbr2.at[c]
    msg_a = sets[0][0]

    def zrow(i, carry):
        for j in range(HALF // 16):
            msg_a[i, pl.ds(j * 16, 16)] = jnp.zeros((16,), jnp.float32)
        return carry
    lax.fori_loop(0, K, zrow, 0)
    for t in range(RPS // K):
        pltpu.sync_copy(msg_a, acc.at[pl.ds(s * RPS + t * K, K)])
    rem = RPS - (RPS // K) * K
    if rem:
        pltpu.sync_copy(msg_a.at[pl.ds(0, rem)],
                        acc.at[pl.ds(s * RPS + (RPS // K) * K, rem)])

    @pl.when(s == NSUB - 1)
    def _():
        pltpu.sync_copy(msg_a.at[pl.ds(0, N - NSUB * RPS)],
                        acc.at[pl.ds(NSUB * RPS, N - NSUB * RPS)])
    plsc.subcore_barrier()

    def start_ci(m, st):
        base = s * EPW + m * K
        pltpu.async_copy(col_hbm.at[pl.ds(base, K)], st[1], st[6])
        pltpu.async_copy(ev_hbm.at[pl.ds(base, K)], st[3], st[6])

    def wait_ci(m, st):
        base = s * EPW + m * K
        pltpu.make_async_copy(col_hbm.at[pl.ds(base, K)], st[1], st[6]).wait()
        pltpu.make_async_copy(ev_hbm.at[pl.ds(base, K)], st[3], st[6]).wait()

    def start_row(m, st):
        pltpu.async_copy(row_hbm.at[pl.ds(s * EPW + m * K, K)], st[2], st[7])

    def wait_row(m, st):
        pltpu.make_async_copy(row_hbm.at[pl.ds(s * EPW + m * K, K)],
                              st[2], st[7]).wait()

    def start_gather(st):
        pltpu.async_copy(nbr_ref.at[st[1]], st[0], st[4])

    def wait_gather(st):
        pltpu.make_async_copy(nbr_ref.at[st[1]], st[0], st[4]).wait()

    def start_scatter(st):
        pltpu.async_copy(st[0], acc.at[st[2]], st[5], add=True)

    def wait_scatter(st):
        pltpu.make_async_copy(st[0], acc.at[st[2]], st[5]).wait()

    def scale(st):
        msg_x, ev_x = st[0], st[3]

        @plsc.parallel_loop(0, K // 16)
        def _(g):
            ev16 = ev_x[pl.ds(g * 16, 16)]
            for l in range(16):
                e = g * 16 + l
                ev_s = ev16[l]
                for j in range(HALF // 16):
                    msg_x[e, pl.ds(j * 16, 16)] = msg_x[e, pl.ds(j * 16, 16)] * ev_s

    def guarded(m, work):
        if isinstance(m, int):
            if m <= CH - 1:
                work()
        else:
            pl.when(m <= CH - 1)(work)

    def phase_at(m, p):
        cur = sets[p]
        far = sets[(p + 2) % NBUF]
        if not (isinstance(m, int) and m < 2):
            wait_scatter(far)

        def ahead():
            start_row(m + 2, far)
            wait_ci(m + 2, far)
            start_gather(far)
        guarded(m + 2, ahead)
        wait_gather(cur)
        scale(cur)
        wait_row(m, cur)
        start_scatter(cur)
        guarded(m + 4, lambda: start_ci(m + 4, cur))

    for j in range(NBUF):
        start_ci(j, sets[j])
    start_row(0, sets[0])
    start_row(1, sets[1])
    wait_ci(0, sets[0])
    start_gather(sets[0])
    wait_ci(1, sets[1])
    start_gather(sets[1])
    phase_at(0, 0)
    phase_at(1, 1)

    def body(i, carry):
        m0 = NBUF * i + 2
        for j in range(NBUF):
            phase_at(m0 + j, (2 + j) % NBUF)
        return carry
    lax.fori_loop(0, (CH - 2) // NBUF, body, 0)
    tail_start = 2 + NBUF * ((CH - 2) // NBUF)
    for m in range(tail_start, CH):
        phase_at(m, m % NBUF)
    wait_scatter(sets[(CH - 2) % NBUF])
    wait_scatter(sets[(CH - 1) % NBUF])

    plsc.subcore_barrier()

    out_ref = out2.at[c]
    pltpu.sync_copy(acc.at[pl.ds(s * RPS, RPS)], out_ref.at[pl.ds(s * RPS, RPS)])

    @pl.when(s == NSUB - 1)
    def _():
        pltpu.sync_copy(acc.at[pl.ds(NSUB * RPS, N - NSUB * RPS)],
                        out_ref.at[pl.ds(NSUB * RPS, N - NSUB * RPS)])


@functools.cache
def _get_sc_aggregate():
  buf_types = []
  for _ in range(NBUF):
      buf_types += [
          pltpu.VMEM((K, HALF), jnp.float32),
          pltpu.VMEM((K,), jnp.int32),
          pltpu.VMEM((K,), jnp.int32),
          pltpu.VMEM((K,), jnp.float32),
      ]
  sem_types = []
  for _ in range(NBUF):
      sem_types += [pltpu.SemaphoreType.DMA] * 4
  return pl.kernel(
    _sc_body,
    out_type=jax.ShapeDtypeStruct((2, N, HALF), jnp.float32),
    mesh=plsc.VectorSubcoreMesh(core_axis_name="c", subcore_axis_name="s"),
    scratch_types=buf_types
    + [pltpu.VMEM_SHARED((N, HALF), jnp.float32)]
    + sem_types,
  )


R = 1000
GRID = N // R


def _rows(width):
    return pl.BlockSpec((R, width), lambda i: (i, 0))


def _half2():
    return pl.BlockSpec((2, R, HALF), lambda i: (0, i, 0))


def _full(shape):
    return pl.BlockSpec(shape, lambda i: (0,) * len(shape))


def _tc_a_body(x_ref, w1, b1, w2, b2, wn, bn, ws, bs,
               h_ref, nb_ref, sf_ref):
    t = _gelu(_dotT(x_ref[...], w1[...]) + b1[...])
    h = _dotT(t, w2[...]) + b2[...]
    h_ref[...] = h
    nbr = _dotT(h, wn[...]) + bn[...]
    nb_ref[0] = nbr[:, :HALF]
    nb_ref[1] = nbr[:, HALF:]
    sf_ref[...] = _dotT(h, ws[...]) + bs[...]


_tc_a = pl.pallas_call(
    _tc_a_body,
    grid=(GRID,),
    in_specs=[_rows(IN), _full((D, IN)), _full((1, D)), _full((D, D)),
              _full((1, D)), _full((D, D)), _full((1, D)), _full((D, D)),
              _full((1, D))],
    out_specs=[_rows(D), _half2(), _rows(D)],
    out_shape=[jax.ShapeDtypeStruct((N, D), jnp.float32),
               jax.ShapeDtypeStruct((2, N, HALF), jnp.float32),
               jax.ShapeDtypeStruct((N, D), jnp.float32)],
)


def _tc_b_body(h_ref, sf_ref, ag_ref, wg1s, wg1l, wg1h, bg1, wg2, bg2,
               wn, bn, ws, bs, h1_ref, nb_ref, s1_ref):
    t = (_dotT(sf_ref[...], wg1s[...]) + _dotT(ag_ref[0], wg1l[...])
         + _dotT(ag_ref[1], wg1h[...]) + bg1[...])
    out = _dotT(_gelu(t), wg2[...]) + bg2[...]
    h1 = h_ref[...] + out
    h1_ref[...] = h1
    nbr = _dotT(h1, wn[...]) + bn[...]
    nb_ref[0] = nbr[:, :HALF]
    nb_ref[1] = nbr[:, HALF:]
    s1_ref[...] = _dotT(h1, ws[...]) + bs[...]


_tc_b = pl.pallas_call(
    _tc_b_body,
    grid=(GRID,),
    in_specs=[_rows(D), _rows(D), _half2(),
              _full((D, D)), _full((D, HALF)), _full((D, HALF)), _full((1, D)),
              _full((D, D)), _full((1, D)),
              _full((D, D)), _full((1, D)), _full((D, D)), _full((1, D))],
    out_specs=[_rows(D), _half2(), _rows(D)],
    out_shape=[jax.ShapeDtypeStruct((N, D), jnp.float32),
               jax.ShapeDtypeStruct((2, N, HALF), jnp.float32),
               jax.ShapeDtypeStruct((N, D), jnp.float32)],
)


def _tc_c_body(h_ref, sf_ref, ag_ref, wg1s, wg1l, wg1h, bg1, wg2, bg2,
               g_ref, bnorm_ref, o_ref):
    t = (_dotT(sf_ref[...], wg1s[...]) + _dotT(ag_ref[0], wg1l[...])
         + _dotT(ag_ref[1], wg1h[...]) + bg1[...])
    out = _dotT(_gelu(t), wg2[...]) + bg2[...]
    hf = h_ref[...] + out
    mu = jnp.mean(hf, axis=-1, keepdims=True)
    var = jnp.mean((hf - mu) ** 2, axis=-1, keepdims=True)
    o_ref[...] = (hf - mu) / jnp.sqrt(var + 1e-5) * g_ref[...] + bnorm_ref[...]


_tc_c = pl.pallas_call(
    _tc_c_body,
    grid=(GRID,),
    in_specs=[_rows(D), _rows(D), _half2(),
              _full((D, D)), _full((D, HALF)), _full((D, HALF)), _full((1, D)),
              _full((D, D)), _full((1, D)),
              _full((1, D)), _full((1, D))],
    out_specs=[_rows(D)],
    out_shape=[jax.ShapeDtypeStruct((N, D), jnp.float32)],
)


def kernel(x, edge_index, edge_values,
           lift_W1, lift_b1, lift_W2, lift_b2,
           gcn0_Ws, gcn0_bs, gcn0_Wn, gcn0_bn, gcn0_Wg1, gcn0_bg1, gcn0_Wg2, gcn0_bg2,
           gcn1_Ws, gcn1_bs, gcn1_Wn, gcn1_bn, gcn1_Wg1, gcn1_bg1, gcn1_Wg2, gcn1_bg2,
           norm_g, norm_b):
    x2 = x.reshape(N, IN)
    row = edge_index[0]
    col = edge_index[1]

    def b(v):
        return v.reshape(1, D)

    h, nb0, sf = _tc_a(x2, lift_W1, b(lift_b1), lift_W2, b(lift_b2),
                       gcn0_Wn, b(gcn0_bn), gcn0_Ws, b(gcn0_bs))
    sc_aggregate = _get_sc_aggregate()
    ag0 = sc_aggregate(nb0, row, col, edge_values)
    h1, nb1, s1 = _tc_b(h, sf, ag0,
                        gcn0_Wg1[:, :D], gcn0_Wg1[:, D:D + HALF],
                        gcn0_Wg1[:, D + HALF:], b(gcn0_bg1),
                        gcn0_Wg2, b(gcn0_bg2),
                        gcn1_Wn, b(gcn1_bn), gcn1_Ws, b(gcn1_bs))
    ag1 = sc_aggregate(nb1, row, col, edge_values)
    (out,) = _tc_c(h1, s1, ag1,
                   gcn1_Wg1[:, :D], gcn1_Wg1[:, D:D + HALF],
                   gcn1_Wg1[:, D + HALF:], b(gcn1_bg1),
                   gcn1_Wg2, b(gcn1_bg2),
                   norm_g.reshape(1, D), norm_b.reshape(1, D))
    return out.reshape(1, N, D)
